# baseline (speedup 1.0000x reference)
import functools

import jax
import jax.numpy as jnp
import numpy as np
from jax import lax
from jax.experimental import pallas as pl
from jax.experimental.pallas import tpu as pltpu
from jax.experimental.pallas import tpu_sc as plsc

ALPHA = 0.5
NUM_THETA = 180
RHO_BINS = 512
LINE_THRESH = 50.0
H = W = 512

_DIAG = float(np.sqrt(2.0) * 512.0)
_K = (RHO_BINS - 1) / (2.0 * _DIAG)
_C = np.float32(_DIAG * _K)

_NPAD = 192
_thetas = np.linspace(-np.pi / 2.0, np.pi / 2.0, NUM_THETA).astype(np.float32)
_A = np.zeros(_NPAD, np.float32)
_B = np.zeros(_NPAD, np.float32)
_A[:NUM_THETA] = (np.cos(_thetas.astype(np.float64)) * _K).astype(np.float32)
_B[:NUM_THETA] = (np.sin(_thetas.astype(np.float64)) * _K).astype(np.float32)
_ATAB = np.repeat(_A[:, None], 16, axis=1).reshape(-1)
_BTAB = np.repeat(_B[:, None], 16, axis=1).reshape(-1)

_NC, _NS = 2, 16
_JOBS = _NPAD // 32
_NGRP = 6
_R = 8
_NCH = H // _R
_RPIX = _R * W
_LS = RHO_BINS + 1
_SUB = 16 * _LS
_ACC_WORDS = 2 * 4 * _SUB


def _hough_sc_body(masks_hbm, atab_hbm, btab_hbm, out_hbm,
                   accv, chunkv, atabv, btabv, histv, sem0, sem1):
    wid = lax.axis_index("s") * _NC + lax.axis_index("c")

    pltpu.sync_copy(atab_hbm, atabv)
    pltpu.sync_copy(btab_hbm, btabv)

    lane = lax.iota(jnp.int32, 16)
    lanef = lane.astype(jnp.float32)
    cvecf = jnp.full((16,), _C, jnp.float32)
    offv = [lane * _LS + s * _SUB for s in range(8)]
    zero16 = jnp.zeros((16,), jnp.float32)

    def group_body(g, _):
        p = g // 2
        hh = g % 2
        mb = hh * 4

        def zb(i, _):
            accv[pl.ds(i * 16, 16)] = zero16
            return 0
        lax.fori_loop(0, _ACC_WORDS // 16, zb, 0)

        def dma(ci, buf, sem):
            return pltpu.make_async_copy(
                masks_hbm.at[pl.ds(mb, 4),
                             pl.ds(pl.multiple_of(ci * _RPIX, 8), _RPIX)],
                chunkv.at[buf], sem)

        dma(0, 0, sem0).start()

        tes, avs, bvs, rmasks = [], [], [], []
        for jj in range(2):
            cidx = wid + 32 * (2 * p + jj)
            real = cidx < NUM_THETA
            te = jnp.where(real, cidx, NUM_THETA)
            tes.append(te)
            avs.append(atabv[pl.ds(pl.multiple_of(te * 16, 8), 16)])
            bvs.append(btabv[pl.ds(pl.multiple_of(te * 16, 8), 16)])
            rmasks.append(jnp.full((16,), real, jnp.bool_))

        def process(ci, buf):
            rb = ci * _R
            for jj in range(2):
                av, bv = avs[jj], bvs[jj]
                base_l = lanef * av + cvecf
                stepv = av * 16.0
                davs = [av * 128.0, av * 256.0, av * 384.0]

                @plsc.parallel_loop(0, _R, 1, unroll=2)
                def row_body(r):
                    yf = jnp.full((16,), rb + r, jnp.int32).astype(jnp.float32)
                    cy = base_l + yf * bv
                    ch = [cy, cy + davs[0], cy + davs[1], cy + davs[2]]
                    rw = r * W

                    def loads(xq):
                        return [chunkv[buf, m,
                                       pl.ds(rw + (xq + 8 * q) * 16, 16)]
                                for q in range(4) for m in range(4)]

                    wvs = loads(0)
                    for xq in range(8):
                        nxt = loads(xq + 1) if xq < 7 else None
                        bins = [ch[q].astype(jnp.int32) for q in range(4)]
                        for q in range(4):
                            for m in range(4):
                                plsc.addupdate_scatter(
                                    accv, [bins[q] + offv[jj * 4 + m]],
                                    wvs[q * 4 + m], mask=rmasks[jj])
                            ch[q] = ch[q] + stepv
                        wvs = nxt

        def pair_body(c2, _):
            ci0 = c2 * 2
            dma(ci0, 0, sem0).wait()

            @pl.when(ci0 + 1 < _NCH)
            def _():
                dma(ci0 + 1, 1, sem1).start()
            process(ci0, 0)

            @pl.when(ci0 + 1 < _NCH)
            def _():
                dma(ci0 + 1, 1, sem1).wait()

                @pl.when(ci0 + 2 < _NCH)
                def _():
                    dma(ci0 + 2, 0, sem0).start()
                process(ci0 + 1, 1)
            return 0
        lax.fori_loop(0, _NCH // 2, pair_body, 0)

        for jj in range(2):
            cidx = wid + 32 * (2 * p + jj)
            real = cidx < NUM_THETA

            @pl.when(real)
            def _(jj=jj, cidx=cidx):
                for m in range(4):
                    sbase = (jj * 4 + m) * _SUB

                    def red(c, _):
                        s = accv[pl.ds(sbase + c * 16, 16)]
                        for l in range(1, 16):
                            s = s + accv[pl.ds(sbase + l * _LS + c * 16, 16)]
                        histv[pl.ds(c * 16, 16)] = s
                        return 0
                    lax.fori_loop(0, RHO_BINS // 16, red, 0)
                    pltpu.sync_copy(histv, out_hbm.at[mb + m, cidx])
        return 0
    lax.fori_loop(0, _NGRP, group_body, 0)


@functools.cache
def _hough_sc():
    return pl.kernel(
        _hough_sc_body,
        out_type=jax.ShapeDtypeStruct((8, NUM_THETA, RHO_BINS), jnp.float32),
        mesh=plsc.VectorSubcoreMesh(core_axis_name="c", subcore_axis_name="s",
                                    num_cores=_NC, num_subcores=_NS),
        compiler_params=pltpu.CompilerParams(needs_layout_passes=False),
        scratch_types=[
            pltpu.VMEM((_ACC_WORDS,), jnp.float32),
            pltpu.VMEM((2, 4, _RPIX), jnp.float32),
            pltpu.VMEM((_NPAD * 16,), jnp.float32),
            pltpu.VMEM((_NPAD * 16,), jnp.float32),
            pltpu.VMEM((RHO_BINS,), jnp.float32),
            pltpu.SemaphoreType.DMA,
            pltpu.SemaphoreType.DMA,
        ],
    )


def _prep_body(lg_ref, tg_ref, mask_ref, sums_ref):
    i = pl.program_id(0)
    lg = lg_ref[0]
    tg = tg_ref[0]
    probs = jax.nn.sigmoid(lg)
    is_pred = i < 4
    mask = jnp.where(is_pred, (lg > 0.0).astype(jnp.float32),
                     (tg > 0.5).astype(jnp.float32))
    mask_ref[0] = mask
    pf = is_pred.astype(jnp.float32)
    s0 = jnp.sum(probs * tg) * pf
    s1 = jnp.sum(probs) * pf
    s2 = jnp.sum(tg) * (1.0 - pf)
    lanes = lax.broadcasted_iota(jnp.int32, (1, 1, 128), 2)
    sums_ref[...] = jnp.where(
        lanes == 0, s0, jnp.where(lanes == 1, s1, jnp.where(lanes == 2, s2, 0.0)))


def _post_body(ap_ref, at_ref, sums_ref):
    ap = ap_ref[0]
    at = at_ref[0]
    tp = jnp.where(ap >= LINE_THRESH, ap, 0.0)
    tt = jnp.where(at >= LINE_THRESH, at, 0.0)
    php = tp / jnp.maximum(jnp.max(tp), 1e-12)
    pht = tt / jnp.maximum(jnp.max(tt), 1e-12)
    s0 = jnp.sum(php * pht)
    s1 = jnp.sum(php)
    s2 = jnp.sum(pht)
    lanes = lax.broadcasted_iota(jnp.int32, (1, 1, 128), 2)
    sums_ref[...] = jnp.where(
        lanes == 0, s0, jnp.where(lanes == 1, s1, jnp.where(lanes == 2, s2, 0.0)))


def kernel(logits, targets):
    lg = logits.reshape(4, H, W)
    tg = targets.reshape(4, H, W)

    masks, sums1 = pl.pallas_call(
        _prep_body,
        grid=(8,),
        in_specs=[
            pl.BlockSpec((1, H, W), lambda i: (i % 4, 0, 0)),
            pl.BlockSpec((1, H, W), lambda i: (i % 4, 0, 0)),
        ],
        out_specs=[
            pl.BlockSpec((1, H, W), lambda i: (i, 0, 0)),
            pl.BlockSpec((1, 1, 128), lambda i: (i, 0, 0)),
        ],
        out_shape=[
            jax.ShapeDtypeStruct((8, H, W), jnp.float32),
            jax.ShapeDtypeStruct((8, 1, 128), jnp.float32),
        ],
    )(lg, tg)

    acc8 = _hough_sc()(masks.reshape(8, H * W),
                       jnp.asarray(_ATAB), jnp.asarray(_BTAB))

    sums3 = pl.pallas_call(
        _post_body,
        grid=(4,),
        in_specs=[
            pl.BlockSpec((1, NUM_THETA, RHO_BINS), lambda i: (i, 0, 0)),
            pl.BlockSpec((1, NUM_THETA, RHO_BINS), lambda i: (i + 4, 0, 0)),
        ],
        out_specs=pl.BlockSpec((1, 1, 128), lambda i: (i, 0, 0)),
        out_shape=jax.ShapeDtypeStruct((4, 1, 128), jnp.float32),
    )(acc8, acc8)

    i1 = jnp.sum(sums1[:, 0, 0])
    card1 = jnp.sum(sums1[:, 0, 1]) + jnp.sum(sums1[:, 0, 2])
    loss_img = 1.0 - 2.0 * i1 / jnp.maximum(card1, 1e-7)

    i2 = jnp.sum(sums3[:, 0, 0])
    card2 = jnp.sum(sums3[:, 0, 1]) + jnp.sum(sums3[:, 0, 2])
    loss_h = 1.0 - 2.0 * i2 / jnp.maximum(card2, 1e-7)

    return ((1.0 - ALPHA) * loss_img + ALPHA * loss_h).astype(jnp.float32)


# speedup vs baseline: 1.0091x; 1.0091x over previous
import functools

import jax
import jax.numpy as jnp
import numpy as np
from jax import lax
from jax.experimental import pallas as pl
from jax.experimental.pallas import tpu as pltpu
from jax.experimental.pallas import tpu_sc as plsc

ALPHA = 0.5
NUM_THETA = 180
RHO_BINS = 512
LINE_THRESH = 50.0
H = W = 512

_DIAG = float(np.sqrt(2.0) * 512.0)
_K = (RHO_BINS - 1) / (2.0 * _DIAG)
_C = np.float32(_DIAG * _K)

_NPAD = 192
_thetas = np.linspace(-np.pi / 2.0, np.pi / 2.0, NUM_THETA).astype(np.float32)
_A = np.zeros(_NPAD, np.float32)
_B = np.zeros(_NPAD, np.float32)
_A[:NUM_THETA] = (np.cos(_thetas.astype(np.float64)) * _K).astype(np.float32)
_B[:NUM_THETA] = (np.sin(_thetas.astype(np.float64)) * _K).astype(np.float32)
_ATAB = np.repeat(_A[:, None], 16, axis=1).reshape(-1)
_BTAB = np.repeat(_B[:, None], 16, axis=1).reshape(-1)

_NC, _NS = 2, 16
_JOBS = _NPAD // 32
_NGRP = 6
_R = 8
_NCH = H // _R
_RPIX = _R * W
_LS = RHO_BINS + 1
_SUB = 16 * _LS
_ACC_WORDS = 2 * 4 * _SUB


def _hough_sc_body(masks_hbm, atab_hbm, btab_hbm, out_hbm,
                   accv, chunkv, atabv, btabv, histv, sem0, sem1):
    wid = lax.axis_index("s") * _NC + lax.axis_index("c")

    pltpu.sync_copy(atab_hbm, atabv)
    pltpu.sync_copy(btab_hbm, btabv)

    lane = lax.iota(jnp.int32, 16)
    lanef = lane.astype(jnp.float32)
    cvecf = jnp.full((16,), _C, jnp.float32)
    offv = [lane * _LS + s * _SUB for s in range(8)]
    zero16 = jnp.zeros((16,), jnp.float32)

    def group_body(g, _):
        p = g // 2
        hh = g % 2
        mb = hh * 4

        def zb(i, _):
            accv[pl.ds(i * 16, 16)] = zero16
            return 0
        lax.fori_loop(0, _ACC_WORDS // 16, zb, 0)

        def dma(ci, buf, sem):
            return pltpu.make_async_copy(
                masks_hbm.at[pl.ds(mb, 4),
                             pl.ds(pl.multiple_of(ci * _RPIX, 8), _RPIX)],
                chunkv.at[buf], sem)

        dma(0, 0, sem0).start()

        tes, avs, bvs, rmasks = [], [], [], []
        for jj in range(2):
            cidx = wid + 32 * (2 * p + jj)
            real = cidx < NUM_THETA
            te = jnp.where(real, cidx, NUM_THETA)
            tes.append(te)
            avs.append(atabv[pl.ds(pl.multiple_of(te * 16, 8), 16)])
            bvs.append(btabv[pl.ds(pl.multiple_of(te * 16, 8), 16)])
            rmasks.append(jnp.full((16,), real, jnp.bool_))

        def process(ci, buf):
            rb = ci * _R
            for jj in range(2):
                av, bv = avs[jj], bvs[jj]
                base_l = lanef * av + cvecf
                stepv = av * 16.0
                davs = [av * 128.0, av * 256.0, av * 384.0]

                @plsc.parallel_loop(0, _R, 1, unroll=1)
                def row_body(r):
                    yf = jnp.full((16,), rb + r, jnp.int32).astype(jnp.float32)
                    cy = base_l + yf * bv
                    ch = [cy, cy + davs[0], cy + davs[1], cy + davs[2]]
                    rw = r * W

                    def loads(xq):
                        return [chunkv[buf, m,
                                       pl.ds(rw + (xq + 8 * q) * 16, 16)]
                                for q in range(4) for m in range(4)]

                    wvs = loads(0)
                    for xq in range(8):
                        nxt = loads(xq + 1) if xq < 7 else None
                        bins = [ch[q].astype(jnp.int32) for q in range(4)]
                        for q in range(4):
                            for m in range(4):
                                plsc.addupdate_scatter(
                                    accv, [bins[q] + offv[jj * 4 + m]],
                                    wvs[q * 4 + m], mask=rmasks[jj])
                            ch[q] = ch[q] + stepv
                        wvs = nxt

        def pair_body(c2, _):
            ci0 = c2 * 2
            dma(ci0, 0, sem0).wait()

            @pl.when(ci0 + 1 < _NCH)
            def _():
                dma(ci0 + 1, 1, sem1).start()
            process(ci0, 0)

            @pl.when(ci0 + 1 < _NCH)
            def _():
                dma(ci0 + 1, 1, sem1).wait()

                @pl.when(ci0 + 2 < _NCH)
                def _():
                    dma(ci0 + 2, 0, sem0).start()
                process(ci0 + 1, 1)
            return 0
        lax.fori_loop(0, _NCH // 2, pair_body, 0)

        for jj in range(2):
            cidx = wid + 32 * (2 * p + jj)
            real = cidx < NUM_THETA

            @pl.when(real)
            def _(jj=jj, cidx=cidx):
                for m in range(4):
                    sbase = (jj * 4 + m) * _SUB

                    def red(c, _):
                        s = accv[pl.ds(sbase + c * 16, 16)]
                        for l in range(1, 16):
                            s = s + accv[pl.ds(sbase + l * _LS + c * 16, 16)]
                        histv[pl.ds(c * 16, 16)] = s
                        return 0
                    lax.fori_loop(0, RHO_BINS // 16, red, 0)
                    pltpu.sync_copy(histv, out_hbm.at[mb + m, cidx])
        return 0
    lax.fori_loop(0, _NGRP, group_body, 0)


@functools.cache
def _hough_sc():
    return pl.kernel(
        _hough_sc_body,
        out_type=jax.ShapeDtypeStruct((8, NUM_THETA, RHO_BINS), jnp.float32),
        mesh=plsc.VectorSubcoreMesh(core_axis_name="c", subcore_axis_name="s",
                                    num_cores=_NC, num_subcores=_NS),
        compiler_params=pltpu.CompilerParams(needs_layout_passes=False),
        scratch_types=[
            pltpu.VMEM((_ACC_WORDS,), jnp.float32),
            pltpu.VMEM((2, 4, _RPIX), jnp.float32),
            pltpu.VMEM((_NPAD * 16,), jnp.float32),
            pltpu.VMEM((_NPAD * 16,), jnp.float32),
            pltpu.VMEM((RHO_BINS,), jnp.float32),
            pltpu.SemaphoreType.DMA,
            pltpu.SemaphoreType.DMA,
        ],
    )


def _prep_body(lg_ref, tg_ref, mask_ref, sums_ref):
    i = pl.program_id(0)
    lg = lg_ref[0]
    tg = tg_ref[0]
    probs = jax.nn.sigmoid(lg)
    is_pred = i < 4
    mask = jnp.where(is_pred, (lg > 0.0).astype(jnp.float32),
                     (tg > 0.5).astype(jnp.float32))
    mask_ref[0] = mask
    pf = is_pred.astype(jnp.float32)
    s0 = jnp.sum(probs * tg) * pf
    s1 = jnp.sum(probs) * pf
    s2 = jnp.sum(tg) * (1.0 - pf)
    lanes = lax.broadcasted_iota(jnp.int32, (1, 1, 128), 2)
    sums_ref[...] = jnp.where(
        lanes == 0, s0, jnp.where(lanes == 1, s1, jnp.where(lanes == 2, s2, 0.0)))


def _post_body(ap_ref, at_ref, sums_ref):
    ap = ap_ref[0]
    at = at_ref[0]
    tp = jnp.where(ap >= LINE_THRESH, ap, 0.0)
    tt = jnp.where(at >= LINE_THRESH, at, 0.0)
    php = tp / jnp.maximum(jnp.max(tp), 1e-12)
    pht = tt / jnp.maximum(jnp.max(tt), 1e-12)
    s0 = jnp.sum(php * pht)
    s1 = jnp.sum(php)
    s2 = jnp.sum(pht)
    lanes = lax.broadcasted_iota(jnp.int32, (1, 1, 128), 2)
    sums_ref[...] = jnp.where(
        lanes == 0, s0, jnp.where(lanes == 1, s1, jnp.where(lanes == 2, s2, 0.0)))


def kernel(logits, targets):
    lg = logits.reshape(4, H, W)
    tg = targets.reshape(4, H, W)

    masks, sums1 = pl.pallas_call(
        _prep_body,
        grid=(8,),
        in_specs=[
            pl.BlockSpec((1, H, W), lambda i: (i % 4, 0, 0)),
            pl.BlockSpec((1, H, W), lambda i: (i % 4, 0, 0)),
        ],
        out_specs=[
            pl.BlockSpec((1, H, W), lambda i: (i, 0, 0)),
            pl.BlockSpec((1, 1, 128), lambda i: (i, 0, 0)),
        ],
        out_shape=[
            jax.ShapeDtypeStruct((8, H, W), jnp.float32),
            jax.ShapeDtypeStruct((8, 1, 128), jnp.float32),
        ],
    )(lg, tg)

    acc8 = _hough_sc()(masks.reshape(8, H * W),
                       jnp.asarray(_ATAB), jnp.asarray(_BTAB))

    sums3 = pl.pallas_call(
        _post_body,
        grid=(4,),
        in_specs=[
            pl.BlockSpec((1, NUM_THETA, RHO_BINS), lambda i: (i, 0, 0)),
            pl.BlockSpec((1, NUM_THETA, RHO_BINS), lambda i: (i + 4, 0, 0)),
        ],
        out_specs=pl.BlockSpec((1, 1, 128), lambda i: (i, 0, 0)),
        out_shape=jax.ShapeDtypeStruct((4, 1, 128), jnp.float32),
    )(acc8, acc8)

    i1 = jnp.sum(sums1[:, 0, 0])
    card1 = jnp.sum(sums1[:, 0, 1]) + jnp.sum(sums1[:, 0, 2])
    loss_img = 1.0 - 2.0 * i1 / jnp.maximum(card1, 1e-7)

    i2 = jnp.sum(sums3[:, 0, 0])
    card2 = jnp.sum(sums3[:, 0, 1]) + jnp.sum(sums3[:, 0, 2])
    loss_h = 1.0 - 2.0 * i2 / jnp.maximum(card2, 1e-7)

    return ((1.0 - ALPHA) * loss_img + ALPHA * loss_h).astype(jnp.float32)


# speedup vs baseline: 2.2272x; 2.2072x over previous
import functools

import jax
import jax.numpy as jnp
import numpy as np
from jax import lax
from jax.experimental import pallas as pl
from jax.experimental.pallas import tpu as pltpu
from jax.experimental.pallas import tpu_sc as plsc

ALPHA = 0.5
NUM_THETA = 180
RHO_BINS = 512
LINE_THRESH = 50.0
H = W = 512

_DIAG = float(np.sqrt(2.0) * 512.0)
_K = (RHO_BINS - 1) / (2.0 * _DIAG)
_C = np.float32(_DIAG * _K)

_NPAD = 192
_thetas = np.linspace(-np.pi / 2.0, np.pi / 2.0, NUM_THETA).astype(np.float32)
_A = np.zeros(_NPAD, np.float32)
_B = np.zeros(_NPAD, np.float32)
_A[:NUM_THETA] = (np.cos(_thetas.astype(np.float64)) * _K).astype(np.float32)
_B[:NUM_THETA] = (np.sin(_thetas.astype(np.float64)) * _K).astype(np.float32)
_ATAB = np.repeat(_A[:, None], 16, axis=1).reshape(-1)
_BTAB = np.repeat(_B[:, None], 16, axis=1).reshape(-1)

_PACK = 4096.0
_NC, _NS = 2, 16
_JOBS = _NPAD // 32
_NGRP = 3
_R = 8
_NCH = H // _R
_RPIX = _R * W
_LS = RHO_BINS + 1
_SUB = 16 * _LS
_ACC_WORDS = 2 * 4 * _SUB


def _hough_sc_body(masks_hbm, atab_hbm, btab_hbm, out_hbm,
                   accv, chunkv, atabv, btabv, histv, sem0, sem1):
    wid = lax.axis_index("s") * _NC + lax.axis_index("c")

    pltpu.sync_copy(atab_hbm, atabv)
    pltpu.sync_copy(btab_hbm, btabv)

    lane = lax.iota(jnp.int32, 16)
    lanef = lane.astype(jnp.float32)
    cvecf = jnp.full((16,), _C, jnp.float32)
    offv = [lane * _LS + s * _SUB for s in range(8)]
    zero16 = jnp.zeros((16,), jnp.float32)

    def group_body(p, _):
        def zb(i, _):
            accv[pl.ds(i * 16, 16)] = zero16
            return 0
        lax.fori_loop(0, _ACC_WORDS // 16, zb, 0)

        def dma(ci, buf, sem):
            return pltpu.make_async_copy(
                masks_hbm.at[pl.ds(0, 4),
                             pl.ds(pl.multiple_of(ci * _RPIX, 8), _RPIX)],
                chunkv.at[buf], sem)

        dma(0, 0, sem0).start()

        tes, avs, bvs, rmasks = [], [], [], []
        for jj in range(2):
            cidx = wid + 32 * (2 * p + jj)
            real = cidx < NUM_THETA
            te = jnp.where(real, cidx, NUM_THETA)
            tes.append(te)
            avs.append(atabv[pl.ds(pl.multiple_of(te * 16, 8), 16)])
            bvs.append(btabv[pl.ds(pl.multiple_of(te * 16, 8), 16)])
            rmasks.append(jnp.full((16,), real, jnp.bool_))

        def process(ci, buf):
            rb = ci * _R
            for jj in range(2):
                av, bv = avs[jj], bvs[jj]
                base_l = lanef * av + cvecf
                stepv = av * 16.0
                davs = [av * 128.0, av * 256.0, av * 384.0]

                def row_body(r, _):
                    yf = jnp.full((16,), rb + r, jnp.int32).astype(jnp.float32)
                    cy = base_l + yf * bv
                    ch = [cy, cy + davs[0], cy + davs[1], cy + davs[2]]
                    rw = r * W

                    def loads(xq):
                        return [chunkv[buf, m,
                                       pl.ds(rw + (xq + 8 * q) * 16, 16)]
                                for q in range(4) for m in range(4)]

                    wvs = loads(0)
                    for xq in range(8):
                        nxt = loads(xq + 1) if xq < 7 else None
                        bins = [ch[q].astype(jnp.int32) for q in range(4)]
                        for q in range(4):
                            for m in range(4):
                                plsc.addupdate_scatter(
                                    accv, [bins[q] + offv[jj * 4 + m]],
                                    wvs[q * 4 + m], mask=rmasks[jj])
                            ch[q] = ch[q] + stepv
                        wvs = nxt
                    return 0
                lax.fori_loop(0, _R, row_body, 0)

        def pair_body(c2, _):
            ci0 = c2 * 2
            dma(ci0, 0, sem0).wait()

            @pl.when(ci0 + 1 < _NCH)
            def _():
                dma(ci0 + 1, 1, sem1).start()
            process(ci0, 0)

            @pl.when(ci0 + 1 < _NCH)
            def _():
                dma(ci0 + 1, 1, sem1).wait()

                @pl.when(ci0 + 2 < _NCH)
                def _():
                    dma(ci0 + 2, 0, sem0).start()
                process(ci0 + 1, 1)
            return 0
        lax.fori_loop(0, _NCH // 2, pair_body, 0)

        inv_pack = jnp.full((16,), 1.0 / _PACK, jnp.float32)
        packv = jnp.full((16,), _PACK, jnp.float32)
        for jj in range(2):
            cidx = wid + 32 * (2 * p + jj)
            real = cidx < NUM_THETA

            @pl.when(real)
            def _(jj=jj, cidx=cidx):
                for m in range(4):
                    sbase = (jj * 4 + m) * _SUB

                    def red(c, _):
                        s = accv[pl.ds(sbase + c * 16, 16)]
                        for l in range(1, 16):
                            s = s + accv[pl.ds(sbase + l * _LS + c * 16, 16)]
                        s1 = (s * inv_pack).astype(jnp.int32).astype(jnp.float32)
                        histv[m, pl.ds(c * 16, 16)] = s - s1 * packv
                        histv[4 + m, pl.ds(c * 16, 16)] = s1
                        return 0
                    lax.fori_loop(0, RHO_BINS // 16, red, 0)
                pltpu.sync_copy(histv, out_hbm.at[cidx])
        return 0
    lax.fori_loop(0, _NGRP, group_body, 0)


@functools.cache
def _hough_sc():
    return pl.kernel(
        _hough_sc_body,
        out_type=jax.ShapeDtypeStruct((NUM_THETA, 8, RHO_BINS), jnp.float32),
        mesh=plsc.VectorSubcoreMesh(core_axis_name="c", subcore_axis_name="s",
                                    num_cores=_NC, num_subcores=_NS),
        compiler_params=pltpu.CompilerParams(needs_layout_passes=False),
        scratch_types=[
            pltpu.VMEM((_ACC_WORDS,), jnp.float32),
            pltpu.VMEM((2, 4, _RPIX), jnp.float32),
            pltpu.VMEM((_NPAD * 16,), jnp.float32),
            pltpu.VMEM((_NPAD * 16,), jnp.float32),
            pltpu.VMEM((8, RHO_BINS), jnp.float32),
            pltpu.SemaphoreType.DMA,
            pltpu.SemaphoreType.DMA,
        ],
    )


def _prep_body(lg_ref, tg_ref, packed_ref, sums_ref):
    lg = lg_ref[0]
    tg = tg_ref[0]
    probs = jax.nn.sigmoid(lg)
    mp = (lg > 0.0).astype(jnp.float32)
    mt = (tg > 0.5).astype(jnp.float32)
    packed_ref[0] = mp + _PACK * mt
    s0 = jnp.sum(probs * tg)
    s1 = jnp.sum(probs)
    s2 = jnp.sum(tg)
    lanes = lax.broadcasted_iota(jnp.int32, (1, 1, 128), 2)
    sums_ref[...] = jnp.where(
        lanes == 0, s0, jnp.where(lanes == 1, s1, jnp.where(lanes == 2, s2, 0.0)))


def _post_body(acc_ref, sums_ref):
    s0 = jnp.float32(0.0)
    s1 = jnp.float32(0.0)
    s2 = jnp.float32(0.0)
    for i in range(4):
        ap = acc_ref[:, i, :]
        at = acc_ref[:, 4 + i, :]
        tp = jnp.where(ap >= LINE_THRESH, ap, 0.0)
        tt = jnp.where(at >= LINE_THRESH, at, 0.0)
        php = tp / jnp.maximum(jnp.max(tp), 1e-12)
        pht = tt / jnp.maximum(jnp.max(tt), 1e-12)
        s0 = s0 + jnp.sum(php * pht)
        s1 = s1 + jnp.sum(php)
        s2 = s2 + jnp.sum(pht)
    lanes = lax.broadcasted_iota(jnp.int32, (1, 1, 128), 2)
    sums_ref[...] = jnp.where(
        lanes == 0, s0, jnp.where(lanes == 1, s1, jnp.where(lanes == 2, s2, 0.0)))


def kernel(logits, targets):
    lg = logits.reshape(4, H, W)
    tg = targets.reshape(4, H, W)

    packed, sums1 = pl.pallas_call(
        _prep_body,
        grid=(4,),
        in_specs=[
            pl.BlockSpec((1, H, W), lambda i: (i, 0, 0)),
            pl.BlockSpec((1, H, W), lambda i: (i, 0, 0)),
        ],
        out_specs=[
            pl.BlockSpec((1, H, W), lambda i: (i, 0, 0)),
            pl.BlockSpec((1, 1, 128), lambda i: (i, 0, 0)),
        ],
        out_shape=[
            jax.ShapeDtypeStruct((4, H, W), jnp.float32),
            jax.ShapeDtypeStruct((4, 1, 128), jnp.float32),
        ],
    )(lg, tg)

    acc = _hough_sc()(packed.reshape(4, H * W),
                      jnp.asarray(_ATAB), jnp.asarray(_BTAB))

    sums3 = pl.pallas_call(
        _post_body,
        grid=(1,),
        in_specs=[
            pl.BlockSpec((NUM_THETA, 8, RHO_BINS), lambda i: (0, 0, 0)),
        ],
        out_specs=pl.BlockSpec((1, 1, 128), lambda i: (0, 0, 0)),
        out_shape=jax.ShapeDtypeStruct((1, 1, 128), jnp.float32),
    )(acc)

    i1 = jnp.sum(sums1[:, 0, 0])
    card1 = jnp.sum(sums1[:, 0, 1]) + jnp.sum(sums1[:, 0, 2])
    loss_img = 1.0 - 2.0 * i1 / jnp.maximum(card1, 1e-7)

    i2 = sums3[0, 0, 0]
    card2 = sums3[0, 0, 1] + sums3[0, 0, 2]
    loss_h = 1.0 - 2.0 * i2 / jnp.maximum(card2, 1e-7)

    return ((1.0 - ALPHA) * loss_img + ALPHA * loss_h).astype(jnp.float32)


# speedup vs baseline: 2.6974x; 1.2111x over previous
import functools

import jax
import jax.numpy as jnp
import numpy as np
from jax import lax
from jax.experimental import pallas as pl
from jax.experimental.pallas import tpu as pltpu
from jax.experimental.pallas import tpu_sc as plsc

ALPHA = 0.5
NUM_THETA = 180
RHO_BINS = 512
LINE_THRESH = 50.0
H = W = 512

_DIAG = float(np.sqrt(2.0) * 512.0)
_K = (RHO_BINS - 1) / (2.0 * _DIAG)
_C = np.float32(_DIAG * _K)

_NPAD = 192
_thetas = np.linspace(-np.pi / 2.0, np.pi / 2.0, NUM_THETA).astype(np.float32)
_A = np.zeros(_NPAD, np.float32)
_B = np.zeros(_NPAD, np.float32)
_A[:NUM_THETA] = (np.cos(_thetas.astype(np.float64)) * _K).astype(np.float32)
_B[:NUM_THETA] = (np.sin(_thetas.astype(np.float64)) * _K).astype(np.float32)
_ATAB = np.repeat(_A[:, None], 16, axis=1).reshape(-1)
_BTAB = np.repeat(_B[:, None], 16, axis=1).reshape(-1)

_PACK = 4096.0
_NC, _NS = 2, 16
_JOBS = _NPAD // 32
_NGRP = 3
_R = 8
_NCH = H // _R
_RPIX = _R * W
_LS = RHO_BINS + 1
_SUB = 16 * _LS
_ACC_WORDS = 2 * 4 * _SUB


def _hough_sc_body(masks_hbm, atab_hbm, btab_hbm, out_hbm,
                   accv, chunkv, atabv, btabv, histv, sem0, sem1):
    wid = lax.axis_index("s") * _NC + lax.axis_index("c")

    pltpu.sync_copy(atab_hbm, atabv)
    pltpu.sync_copy(btab_hbm, btabv)

    lane = lax.iota(jnp.int32, 16)
    lanef = lane.astype(jnp.float32)
    cvecf = jnp.full((16,), _C, jnp.float32)
    offv = [lane * _LS + s * _SUB for s in range(8)]
    zero16 = jnp.zeros((16,), jnp.float32)

    def group_body(p, _):
        def zb(i, _):
            accv[pl.ds(i * 16, 16)] = zero16
            return 0
        lax.fori_loop(0, _ACC_WORDS // 16, zb, 0)

        def dma(ci, buf, sem):
            return pltpu.make_async_copy(
                masks_hbm.at[pl.ds(0, 4),
                             pl.ds(pl.multiple_of(ci * _RPIX, 8), _RPIX)],
                chunkv.at[buf], sem)

        dma(0, 0, sem0).start()

        tes, avs, bvs, rmasks = [], [], [], []
        for jj in range(2):
            cidx = wid + 32 * (2 * p + jj)
            real = cidx < NUM_THETA
            te = jnp.where(real, cidx, NUM_THETA)
            tes.append(te)
            avs.append(atabv[pl.ds(pl.multiple_of(te * 16, 8), 16)])
            bvs.append(btabv[pl.ds(pl.multiple_of(te * 16, 8), 16)])
            rmasks.append(jnp.full((16,), real, jnp.bool_))

        def process(ci, buf):
            rb = ci * _R
            base_ls = [lanef * avs[jj] + cvecf for jj in range(2)]
            stepvs = [avs[jj] * 16.0 for jj in range(2)]
            davs = [[avs[jj] * 128.0, avs[jj] * 256.0, avs[jj] * 384.0]
                    for jj in range(2)]

            def row_body(r, _):
                yf = jnp.full((16,), rb + r, jnp.int32).astype(jnp.float32)
                cys = [base_ls[jj] + yf * bvs[jj] for jj in range(2)]
                ch = [[cys[jj], cys[jj] + davs[jj][0], cys[jj] + davs[jj][1],
                       cys[jj] + davs[jj][2]] for jj in range(2)]
                rw = r * W
                for xq in range(8):
                    wvs = [chunkv[buf, m, pl.ds(rw + (xq + 8 * q) * 16, 16)]
                           for q in range(4) for m in range(4)]
                    bins = [[ch[jj][q].astype(jnp.int32) for q in range(4)]
                            for jj in range(2)]
                    for q in range(4):
                        for m in range(4):
                            for jj in range(2):
                                plsc.addupdate_scatter(
                                    accv, [bins[jj][q] + offv[jj * 4 + m]],
                                    wvs[q * 4 + m], mask=rmasks[jj])
                        for jj in range(2):
                            ch[jj][q] = ch[jj][q] + stepvs[jj]
                return 0
            lax.fori_loop(0, _R, row_body, 0)

        def pair_body(c2, _):
            ci0 = c2 * 2
            dma(ci0, 0, sem0).wait()

            @pl.when(ci0 + 1 < _NCH)
            def _():
                dma(ci0 + 1, 1, sem1).start()
            process(ci0, 0)

            @pl.when(ci0 + 1 < _NCH)
            def _():
                dma(ci0 + 1, 1, sem1).wait()

                @pl.when(ci0 + 2 < _NCH)
                def _():
                    dma(ci0 + 2, 0, sem0).start()
                process(ci0 + 1, 1)
            return 0
        lax.fori_loop(0, _NCH // 2, pair_body, 0)

        inv_pack = jnp.full((16,), 1.0 / _PACK, jnp.float32)
        packv = jnp.full((16,), _PACK, jnp.float32)
        for jj in range(2):
            cidx = wid + 32 * (2 * p + jj)
            real = cidx < NUM_THETA

            @pl.when(real)
            def _(jj=jj, cidx=cidx):
                for m in range(4):
                    sbase = (jj * 4 + m) * _SUB

                    def red(c, _):
                        s = accv[pl.ds(sbase + c * 16, 16)]
                        for l in range(1, 16):
                            s = s + accv[pl.ds(sbase + l * _LS + c * 16, 16)]
                        s1 = (s * inv_pack).astype(jnp.int32).astype(jnp.float32)
                        histv[m, pl.ds(c * 16, 16)] = s - s1 * packv
                        histv[4 + m, pl.ds(c * 16, 16)] = s1
                        return 0
                    lax.fori_loop(0, RHO_BINS // 16, red, 0)
                pltpu.sync_copy(histv, out_hbm.at[cidx])
        return 0
    lax.fori_loop(0, _NGRP, group_body, 0)


@functools.cache
def _hough_sc():
    return pl.kernel(
        _hough_sc_body,
        out_type=jax.ShapeDtypeStruct((NUM_THETA, 8, RHO_BINS), jnp.float32),
        mesh=plsc.VectorSubcoreMesh(core_axis_name="c", subcore_axis_name="s",
                                    num_cores=_NC, num_subcores=_NS),
        compiler_params=pltpu.CompilerParams(needs_layout_passes=False),
        scratch_types=[
            pltpu.VMEM((_ACC_WORDS,), jnp.float32),
            pltpu.VMEM((2, 4, _RPIX), jnp.float32),
            pltpu.VMEM((_NPAD * 16,), jnp.float32),
            pltpu.VMEM((_NPAD * 16,), jnp.float32),
            pltpu.VMEM((8, RHO_BINS), jnp.float32),
            pltpu.SemaphoreType.DMA,
            pltpu.SemaphoreType.DMA,
        ],
    )


def _prep_body(lg_ref, tg_ref, packed_ref, sums_ref):
    lg = lg_ref[0]
    tg = tg_ref[0]
    probs = jax.nn.sigmoid(lg)
    mp = (lg > 0.0).astype(jnp.float32)
    mt = (tg > 0.5).astype(jnp.float32)
    packed_ref[0] = mp + _PACK * mt
    s0 = jnp.sum(probs * tg)
    s1 = jnp.sum(probs)
    s2 = jnp.sum(tg)
    lanes = lax.broadcasted_iota(jnp.int32, (1, 1, 128), 2)
    sums_ref[...] = jnp.where(
        lanes == 0, s0, jnp.where(lanes == 1, s1, jnp.where(lanes == 2, s2, 0.0)))


def _post_body(acc_ref, sums_ref):
    s0 = jnp.float32(0.0)
    s1 = jnp.float32(0.0)
    s2 = jnp.float32(0.0)
    for i in range(4):
        ap = acc_ref[:, i, :]
        at = acc_ref[:, 4 + i, :]
        tp = jnp.where(ap >= LINE_THRESH, ap, 0.0)
        tt = jnp.where(at >= LINE_THRESH, at, 0.0)
        php = tp / jnp.maximum(jnp.max(tp), 1e-12)
        pht = tt / jnp.maximum(jnp.max(tt), 1e-12)
        s0 = s0 + jnp.sum(php * pht)
        s1 = s1 + jnp.sum(php)
        s2 = s2 + jnp.sum(pht)
    lanes = lax.broadcasted_iota(jnp.int32, (1, 1, 128), 2)
    sums_ref[...] = jnp.where(
        lanes == 0, s0, jnp.where(lanes == 1, s1, jnp.where(lanes == 2, s2, 0.0)))


def kernel(logits, targets):
    lg = logits.reshape(4, H, W)
    tg = targets.reshape(4, H, W)

    packed, sums1 = pl.pallas_call(
        _prep_body,
        grid=(4,),
        in_specs=[
            pl.BlockSpec((1, H, W), lambda i: (i, 0, 0)),
            pl.BlockSpec((1, H, W), lambda i: (i, 0, 0)),
        ],
        out_specs=[
            pl.BlockSpec((1, H, W), lambda i: (i, 0, 0)),
            pl.BlockSpec((1, 1, 128), lambda i: (i, 0, 0)),
        ],
        out_shape=[
            jax.ShapeDtypeStruct((4, H, W), jnp.float32),
            jax.ShapeDtypeStruct((4, 1, 128), jnp.float32),
        ],
    )(lg, tg)

    acc = _hough_sc()(packed.reshape(4, H * W),
                      jnp.asarray(_ATAB), jnp.asarray(_BTAB))

    sums3 = pl.pallas_call(
        _post_body,
        grid=(1,),
        in_specs=[
            pl.BlockSpec((NUM_THETA, 8, RHO_BINS), lambda i: (0, 0, 0)),
        ],
        out_specs=pl.BlockSpec((1, 1, 128), lambda i: (0, 0, 0)),
        out_shape=jax.ShapeDtypeStruct((1, 1, 128), jnp.float32),
    )(acc)

    i1 = jnp.sum(sums1[:, 0, 0])
    card1 = jnp.sum(sums1[:, 0, 1]) + jnp.sum(sums1[:, 0, 2])
    loss_img = 1.0 - 2.0 * i1 / jnp.maximum(card1, 1e-7)

    i2 = sums3[0, 0, 0]
    card2 = sums3[0, 0, 1] + sums3[0, 0, 2]
    loss_h = 1.0 - 2.0 * i2 / jnp.maximum(card2, 1e-7)

    return ((1.0 - ALPHA) * loss_img + ALPHA * loss_h).astype(jnp.float32)
